# Initial kernel scaffold; baseline (speedup 1.0000x reference)
import jax
import jax.numpy as jnp
from jax.experimental import pallas as pl


def kernel(x_prime, Wr, fc1_w, fc1_b, fc2_w, fc2_b):
    raise NotImplementedError("write your pallas kernel here")


# speedup vs baseline: 2.2686x; 2.2686x over previous
import functools

import jax
import jax.numpy as jnp
from jax.experimental import pallas as pl
from jax.experimental.pallas import tpu as pltpu

CAP_FACTOR = 2.0


def _router_body(x_ref, wr_ref, aff_ref):
    x = x_ref[0]
    wr = wr_ref[...]
    logits = jax.lax.dot_general(x, wr, (((1,), (1,)), ((), ())),
                                 preferred_element_type=jnp.float32)
    m = jnp.max(logits, axis=1, keepdims=True)
    p = jnp.exp(logits - m)
    aff_ref[0] = p / jnp.sum(p, axis=1, keepdims=True)


def _ffn_body(x_ref, w1_ref, b1_ref, w2_ref, b2_ref, gate_ref, out_ref):
    h_idx = pl.program_id(2)
    x = x_ref[0, 0]
    w1 = w1_ref[0]
    b1 = b1_ref[0]
    h = jax.lax.dot_general(x, w1, (((1,), (1,)), ((), ())),
                            preferred_element_type=jnp.float32)
    h = h + b1
    h = 0.5 * h * (1.0 + jax.lax.erf(h * 0.7071067811865476))
    gate = gate_ref[0, 0]
    h = h * gate.reshape(-1, 1)
    w2 = w2_ref[0]
    y = jax.lax.dot_general(h, w2, (((1,), (1,)), ((), ())),
                            preferred_element_type=jnp.float32)

    @pl.when(h_idx == 0)
    def _():
        out_ref[0, 0] = y + gate.reshape(-1, 1) * b2_ref[0]

    @pl.when(h_idx > 0)
    def _():
        out_ref[0, 0] += y


def kernel(x_prime, Wr, fc1_w, fc1_b, fc2_w, fc2_b):
    B, S, D = x_prime.shape
    E, H, _ = fc1_w.shape
    cap = max(1, int(S * CAP_FACTOR / E) + 1)
    capp = -(-cap // 8) * 8

    St = min(512, S)
    aff = pl.pallas_call(
        _router_body,
        grid=(B, S // St),
        in_specs=[
            pl.BlockSpec((1, St, D), lambda b, s: (b, s, 0)),
            pl.BlockSpec((E, D), lambda b, s: (0, 0)),
        ],
        out_specs=pl.BlockSpec((1, St, E), lambda b, s: (b, s, 0)),
        out_shape=jax.ShapeDtypeStruct((B, S, E), jnp.float32),
    )(x_prime, Wr)

    aff_t = jnp.transpose(aff, (0, 2, 1))
    topv, topi = jax.lax.top_k(aff_t, cap)
    pad = capp - cap
    idxp = jnp.concatenate(
        [topi, jnp.zeros((B, E, pad), topi.dtype)], axis=-1)
    gatep = jnp.concatenate(
        [topv, jnp.zeros((B, E, pad), topv.dtype)], axis=-1)

    xg = x_prime[jnp.arange(B)[:, None, None], idxp]

    Ht = min(512, H)
    yg = pl.pallas_call(
        _ffn_body,
        grid=(B, E, H // Ht),
        in_specs=[
            pl.BlockSpec((1, 1, capp, D), lambda b, e, h: (b, e, 0, 0)),
            pl.BlockSpec((1, Ht, D), lambda b, e, h: (e, h, 0)),
            pl.BlockSpec((1, 1, Ht), lambda b, e, h: (e, 0, h)),
            pl.BlockSpec((1, D, Ht), lambda b, e, h: (e, 0, h)),
            pl.BlockSpec((1, 1, D), lambda b, e, h: (e, 0, 0)),
            pl.BlockSpec((1, 1, 1, capp), lambda b, e, h: (b, e, 0, 0)),
        ],
        out_specs=pl.BlockSpec((1, 1, capp, D), lambda b, e, h: (b, e, 0, 0)),
        out_shape=jax.ShapeDtypeStruct((B, E, capp, D), jnp.float32),
        compiler_params=pltpu.CompilerParams(
            dimension_semantics=("parallel", "parallel", "arbitrary")),
    )(xg, fc1_w, fc1_b.reshape(E, 1, H), fc2_w, fc2_b.reshape(E, 1, D),
      gatep.reshape(B, E, 1, capp))

    out = jnp.zeros_like(x_prime)
    out = out.at[jnp.arange(B)[:, None, None], idxp].add(yg)
    return out


# speedup vs baseline: 2.4302x; 1.0712x over previous
import functools

import jax
import jax.numpy as jnp
from jax.experimental import pallas as pl
from jax.experimental.pallas import tpu as pltpu

CAP_FACTOR = 2.0


def _router_body(x_ref, wr_ref, aff_ref):
    x = x_ref[0]
    wr = wr_ref[...]
    logits = jax.lax.dot_general(x, wr, (((1,), (1,)), ((), ())),
                                 preferred_element_type=jnp.float32)
    m = jnp.max(logits, axis=1, keepdims=True)
    p = jnp.exp(logits - m)
    aff_ref[0] = p / jnp.sum(p, axis=1, keepdims=True)


def _ffn_body(x_ref, w1_ref, b1_ref, w2_ref, b2_ref, gate_ref, out_ref):
    h_idx = pl.program_id(2)
    x = x_ref[0, 0].astype(jnp.bfloat16)
    w1 = w1_ref[0].astype(jnp.bfloat16)
    b1 = b1_ref[0]
    h = jax.lax.dot_general(x, w1, (((1,), (1,)), ((), ())),
                            preferred_element_type=jnp.float32)
    h = h + b1
    h = 0.5 * h * (1.0 + jax.lax.erf(h * 0.7071067811865476))
    gate = gate_ref[0, 0]
    h = (h * gate.reshape(-1, 1)).astype(jnp.bfloat16)
    w2 = w2_ref[0].astype(jnp.bfloat16)
    y = jax.lax.dot_general(h, w2, (((1,), (1,)), ((), ())),
                            preferred_element_type=jnp.float32)

    @pl.when(h_idx == 0)
    def _():
        out_ref[0, 0] = y + gate.reshape(-1, 1) * b2_ref[0]

    @pl.when(h_idx > 0)
    def _():
        out_ref[0, 0] += y


def kernel(x_prime, Wr, fc1_w, fc1_b, fc2_w, fc2_b):
    B, S, D = x_prime.shape
    E, H, _ = fc1_w.shape
    cap = max(1, int(S * CAP_FACTOR / E) + 1)
    capp = -(-cap // 8) * 8

    St = min(512, S)
    aff = pl.pallas_call(
        _router_body,
        grid=(B, S // St),
        in_specs=[
            pl.BlockSpec((1, St, D), lambda b, s: (b, s, 0)),
            pl.BlockSpec((E, D), lambda b, s: (0, 0)),
        ],
        out_specs=pl.BlockSpec((1, St, E), lambda b, s: (b, s, 0)),
        out_shape=jax.ShapeDtypeStruct((B, S, E), jnp.float32),
    )(x_prime, Wr)

    aff_t = jnp.transpose(aff, (0, 2, 1))
    topv, topi = jax.lax.top_k(aff_t, cap)
    pad = capp - cap
    idxp = jnp.concatenate(
        [topi, jnp.zeros((B, E, pad), topi.dtype)], axis=-1)
    gatep = jnp.concatenate(
        [topv, jnp.zeros((B, E, pad), topv.dtype)], axis=-1)

    xg = x_prime[jnp.arange(B)[:, None, None], idxp]

    Ht = min(1024, H)
    yg = pl.pallas_call(
        _ffn_body,
        grid=(B, E, H // Ht),
        in_specs=[
            pl.BlockSpec((1, 1, capp, D), lambda b, e, h: (b, e, 0, 0)),
            pl.BlockSpec((1, Ht, D), lambda b, e, h: (e, h, 0)),
            pl.BlockSpec((1, 1, Ht), lambda b, e, h: (e, 0, h)),
            pl.BlockSpec((1, D, Ht), lambda b, e, h: (e, 0, h)),
            pl.BlockSpec((1, 1, D), lambda b, e, h: (e, 0, 0)),
            pl.BlockSpec((1, 1, 1, capp), lambda b, e, h: (b, e, 0, 0)),
        ],
        out_specs=pl.BlockSpec((1, 1, capp, D), lambda b, e, h: (b, e, 0, 0)),
        out_shape=jax.ShapeDtypeStruct((B, E, capp, D), jnp.float32),
        compiler_params=pltpu.CompilerParams(
            dimension_semantics=("parallel", "parallel", "arbitrary")),
    )(xg, fc1_w, fc1_b.reshape(E, 1, H), fc2_w, fc2_b.reshape(E, 1, D),
      gatep.reshape(B, E, 1, capp))

    out = jnp.zeros_like(x_prime)
    out = out.at[jnp.arange(B)[:, None, None], idxp].add(yg)
    return out
